# baseline (speedup 1.0000x reference)
import functools

import jax
import jax.numpy as jnp
from jax.experimental import pallas as pl
from jax.experimental.pallas import tpu as pltpu

R = 256
CH = 32


def _prep_body(es_ref, w_ref, u_ref, v_ref, sw_ref, uv_ref):
    sw_ref[...] = jnp.dot(es_ref[...], w_ref[...],
                          preferred_element_type=jnp.float32)
    uv_ref[...] = u_ref[...] + v_ref[...]


def _main_body(starts_ref, sidx_ref, h_ref, k_ref, es_ref, sw_ref, uv_ref,
               s_ref, g_ref, t_ref, out_ref, hk_scr, *,
               blk_rows, ents, dim, chunk):
    b = pl.program_id(0)
    h = h_ref[...]
    hk_scr[...] = h + k_ref[...]
    out_ref[...] = h
    s0 = starts_ref[b]
    s1 = starts_ref[b + 1]
    nchunks = (s1 - s0 + chunk - 1) // chunk
    uv = uv_ref[...]
    riota = jax.lax.broadcasted_iota(jnp.int32, (chunk, blk_rows), 1)

    def body(j, _):
        s = s0 + j * chunk
        sv = sidx_ref[pl.ds(s, chunk), :]
        p = (sv - b * blk_rows == riota).astype(jnp.float32)
        hg = jnp.dot(p, h_ref[...],
                     preferred_element_type=jnp.float32)
        hkg = jnp.dot(p, hk_scr[...],
                      preferred_element_type=jnp.float32)
        es_c = es_ref[pl.ds(s, chunk), :]
        es_rep = jnp.dot(es_c, t_ref[...],
                         preferred_element_type=jnp.float32)
        g = jnp.dot(hkg * es_rep, s_ref[...],
                    preferred_element_type=jnp.float32)
        gate_rep = jnp.dot(jax.nn.sigmoid(g), g_ref[...],
                           preferred_element_type=jnp.float32)
        huv = jax.lax.dot_general(
            hg.reshape(chunk, ents, dim), uv,
            (((2,), (0,)), ((), ())),
            preferred_element_type=jnp.float32)
        sw_rep = jnp.dot(sw_ref[pl.ds(s, chunk), :], t_ref[...],
                         preferred_element_type=jnp.float32)
        ht = jnp.maximum(huv.reshape(chunk, ents * dim) + sw_rep, 0.0)
        upd = gate_rep * ht
        out_ref[...] = out_ref[...] + jax.lax.dot_general(
            p, upd, (((0,), (0,)), ((), ())),
            preferred_element_type=jnp.float32)
        return 0

    jax.lax.fori_loop(0, nchunks, body, 0)

    o = out_ref[...]
    ssq = jnp.dot(o * o, s_ref[...],
                  preferred_element_type=jnp.float32)
    rs = jax.lax.rsqrt(jnp.maximum(ssq, 1e-12))
    rs_rep = jnp.dot(rs, g_ref[...],
                     preferred_element_type=jnp.float32)
    out_ref[...] = o * rs_rep


def kernel(hiddens, entity_keys, encoded_sents, U, V, W, indices):
    B, E, D = hiddens.shape
    C = encoded_sents.shape[0]
    nb = B // R
    ED = E * D

    h2 = hiddens.reshape(B, ED)
    k2 = entity_keys.reshape(B, ED)

    idx = indices.astype(jnp.int32)
    sidx, perm = jax.lax.sort_key_val(idx, jnp.arange(C, dtype=jnp.int32))
    es_sorted = jnp.take(encoded_sents, perm, axis=0)
    bounds = jnp.arange(nb + 1, dtype=jnp.int32) * R
    starts = jnp.searchsorted(sidx, bounds).astype(jnp.int32)

    Cpad = C + CH
    sidx_pad = jnp.full((Cpad, 1), B, dtype=jnp.int32)
    sidx_pad = jax.lax.dynamic_update_slice(sidx_pad, sidx[:, None], (0, 0))
    es_pad = jnp.zeros((Cpad, D), jnp.float32)
    es_pad = jax.lax.dynamic_update_slice(es_pad, es_sorted, (0, 0))

    S = jnp.repeat(jnp.eye(E, dtype=jnp.float32), D, axis=0)
    G = S.T
    T = jnp.tile(jnp.eye(D, dtype=jnp.float32), (1, E))

    sw, uv = pl.pallas_call(
        _prep_body,
        out_shape=(
            jax.ShapeDtypeStruct((Cpad, D), jnp.float32),
            jax.ShapeDtypeStruct((D, D), jnp.float32),
        ),
    )(es_pad, W, U, V)

    body = functools.partial(_main_body, blk_rows=R, ents=E, dim=D, chunk=CH)
    grid_spec = pltpu.PrefetchScalarGridSpec(
        num_scalar_prefetch=1,
        grid=(nb,),
        in_specs=[
            pl.BlockSpec((Cpad, 1), lambda b, *_: (0, 0)),
            pl.BlockSpec((R, ED), lambda b, *_: (b, 0)),
            pl.BlockSpec((R, ED), lambda b, *_: (b, 0)),
            pl.BlockSpec((Cpad, D), lambda b, *_: (0, 0)),
            pl.BlockSpec((Cpad, D), lambda b, *_: (0, 0)),
            pl.BlockSpec((D, D), lambda b, *_: (0, 0)),
            pl.BlockSpec((ED, E), lambda b, *_: (0, 0)),
            pl.BlockSpec((E, ED), lambda b, *_: (0, 0)),
            pl.BlockSpec((D, ED), lambda b, *_: (0, 0)),
        ],
        out_specs=pl.BlockSpec((R, ED), lambda b, *_: (b, 0)),
        scratch_shapes=[
            pltpu.VMEM((R, ED), jnp.float32),
        ],
    )
    out = pl.pallas_call(
        body,
        grid_spec=grid_spec,
        out_shape=jax.ShapeDtypeStruct((B, ED), jnp.float32),
        compiler_params=pltpu.CompilerParams(
            dimension_semantics=("arbitrary",)),
    )(starts, sidx_pad, h2, k2, es_pad, sw, uv, S, G, T)
    return out.reshape(B, E, D)


# speedup vs baseline: 1.2222x; 1.2222x over previous
import functools

import jax
import jax.numpy as jnp
from jax.experimental import pallas as pl
from jax.experimental.pallas import tpu as pltpu

R = 128
CH = 64


def _prep_body(es_ref, w_ref, u_ref, v_ref, sw_ref, uv_ref):
    sw_ref[...] = jnp.dot(es_ref[...], w_ref[...],
                          preferred_element_type=jnp.float32)
    uv_ref[...] = u_ref[...] + v_ref[...]


def _main_body(starts_ref, sidx_ref, h_ref, k_ref, es_ref, sw_ref, uv_ref,
               s_ref, g_ref, t_ref, out_ref, hk_scr, *,
               blk_rows, ents, dim, chunk):
    b = pl.program_id(0)
    h = h_ref[...]
    hk_scr[...] = h + k_ref[...]
    out_ref[...] = h
    s0 = starts_ref[b]
    s1 = starts_ref[b + 1]
    nchunks = (s1 - s0 + chunk - 1) // chunk
    uv = uv_ref[...]
    riota = jax.lax.broadcasted_iota(jnp.int32, (chunk, blk_rows), 1)

    def body(j, _):
        s = s0 + j * chunk
        sv = sidx_ref[pl.ds(s, chunk), :]
        p = (sv - b * blk_rows == riota).astype(jnp.float32)
        hg = jnp.dot(p, h_ref[...],
                     preferred_element_type=jnp.float32)
        hkg = jnp.dot(p, hk_scr[...],
                      preferred_element_type=jnp.float32)
        es_c = es_ref[pl.ds(s, chunk), :]
        es_rep = jnp.dot(es_c, t_ref[...],
                         preferred_element_type=jnp.float32)
        g = jnp.dot(hkg * es_rep, s_ref[...],
                    preferred_element_type=jnp.float32)
        gate_rep = jnp.dot(jax.nn.sigmoid(g), g_ref[...],
                           preferred_element_type=jnp.float32)
        huv = jax.lax.dot_general(
            hg.reshape(chunk, ents, dim), uv,
            (((2,), (0,)), ((), ())),
            preferred_element_type=jnp.float32)
        sw_rep = jnp.dot(sw_ref[pl.ds(s, chunk), :], t_ref[...],
                         preferred_element_type=jnp.float32)
        ht = jnp.maximum(huv.reshape(chunk, ents * dim) + sw_rep, 0.0)
        upd = gate_rep * ht
        out_ref[...] = out_ref[...] + jax.lax.dot_general(
            p, upd, (((0,), (0,)), ((), ())),
            preferred_element_type=jnp.float32)
        return 0

    jax.lax.fori_loop(0, nchunks, body, 0)

    o = out_ref[...]
    ssq = jnp.dot(o * o, s_ref[...],
                  preferred_element_type=jnp.float32)
    rs = jax.lax.rsqrt(jnp.maximum(ssq, 1e-12))
    rs_rep = jnp.dot(rs, g_ref[...],
                     preferred_element_type=jnp.float32)
    out_ref[...] = o * rs_rep


def kernel(hiddens, entity_keys, encoded_sents, U, V, W, indices):
    B, E, D = hiddens.shape
    C = encoded_sents.shape[0]
    nb = B // R
    ED = E * D

    h2 = hiddens.reshape(B, ED)
    k2 = entity_keys.reshape(B, ED)

    idx = indices.astype(jnp.int32)
    sidx, perm = jax.lax.sort_key_val(idx, jnp.arange(C, dtype=jnp.int32))
    es_sorted = jnp.take(encoded_sents, perm, axis=0)
    bounds = jnp.arange(nb + 1, dtype=jnp.int32) * R
    starts = jnp.searchsorted(sidx, bounds).astype(jnp.int32)

    Cpad = C + CH
    sidx_pad = jnp.full((Cpad, 1), B, dtype=jnp.int32)
    sidx_pad = jax.lax.dynamic_update_slice(sidx_pad, sidx[:, None], (0, 0))
    es_pad = jnp.zeros((Cpad, D), jnp.float32)
    es_pad = jax.lax.dynamic_update_slice(es_pad, es_sorted, (0, 0))

    S = jnp.repeat(jnp.eye(E, dtype=jnp.float32), D, axis=0)
    G = S.T
    T = jnp.tile(jnp.eye(D, dtype=jnp.float32), (1, E))

    sw, uv = pl.pallas_call(
        _prep_body,
        out_shape=(
            jax.ShapeDtypeStruct((Cpad, D), jnp.float32),
            jax.ShapeDtypeStruct((D, D), jnp.float32),
        ),
    )(es_pad, W, U, V)

    body = functools.partial(_main_body, blk_rows=R, ents=E, dim=D, chunk=CH)
    grid_spec = pltpu.PrefetchScalarGridSpec(
        num_scalar_prefetch=1,
        grid=(nb,),
        in_specs=[
            pl.BlockSpec((Cpad, 1), lambda b, *_: (0, 0)),
            pl.BlockSpec((R, ED), lambda b, *_: (b, 0)),
            pl.BlockSpec((R, ED), lambda b, *_: (b, 0)),
            pl.BlockSpec((Cpad, D), lambda b, *_: (0, 0)),
            pl.BlockSpec((Cpad, D), lambda b, *_: (0, 0)),
            pl.BlockSpec((D, D), lambda b, *_: (0, 0)),
            pl.BlockSpec((ED, E), lambda b, *_: (0, 0)),
            pl.BlockSpec((E, ED), lambda b, *_: (0, 0)),
            pl.BlockSpec((D, ED), lambda b, *_: (0, 0)),
        ],
        out_specs=pl.BlockSpec((R, ED), lambda b, *_: (b, 0)),
        scratch_shapes=[
            pltpu.VMEM((R, ED), jnp.float32),
        ],
    )
    out = pl.pallas_call(
        body,
        grid_spec=grid_spec,
        out_shape=jax.ShapeDtypeStruct((B, ED), jnp.float32),
        compiler_params=pltpu.CompilerParams(
            dimension_semantics=("arbitrary",)),
    )(starts, sidx_pad, h2, k2, es_pad, sw, uv, S, G, T)
    return out.reshape(B, E, D)
